# Initial kernel scaffold; baseline (speedup 1.0000x reference)
import jax
import jax.numpy as jnp
from jax.experimental import pallas as pl


def kernel(total_items, total_adj, embedding, a_0, a_1, a_2, a_3):
    raise NotImplementedError("write your pallas kernel here")


# speedup vs baseline: 1.0877x; 1.0877x over previous
import jax
import jax.numpy as jnp
from jax.experimental import pallas as pl
from jax.experimental.pallas import tpu as pltpu
from jax.experimental.pallas import tpu_sc as plsc

B, L, D = 1024, 50, 128
NEG_SLOPE = 0.2
MASK_VAL = -9e15
BB = 8
GATHER_W = 128


def _sc_gather(emb, idx_flat):
    n = idx_flat.shape[1]
    mesh = plsc.VectorSubcoreMesh(core_axis_name="core",
                                  subcore_axis_name="subcore")

    @pl.kernel(out_type=jax.ShapeDtypeStruct((n, D), emb.dtype), mesh=mesh)
    def gather_kernel(emb_hbm, i_hbm, o_hbm):
        def body(i_vmem, o_vmem):
            pltpu.sync_copy(emb_hbm.at[i_vmem.at[0]], o_vmem)

        pltpu.emit_pipeline(
            body,
            grid=(n // GATHER_W,),
            in_specs=[pl.BlockSpec((1, GATHER_W), lambda i: (0, i))],
            out_specs=[pl.BlockSpec((GATHER_W, D), lambda i: (i, 0))],
            core_axis_name=("core", "subcore"),
            dimension_semantics=(pltpu.PARALLEL,),
        )(i_hbm, o_hbm)

    return gather_kernel(emb, idx_flat)


def _attn_body(a4_ref, h_ref, adj_ref, out_ref):
    a4 = a4_ref[...].astype(jnp.bfloat16)
    for b in range(BB):
        hb = h_ref[b].astype(jnp.bfloat16)
        adjb = adj_ref[b]
        acc = jnp.full((L, L), MASK_VAL, dtype=jnp.float32)
        for k in range(4):
            g = hb * a4[k:k + 1, :]
            e = jax.lax.dot_general(
                hb, g, (((1,), (1,)), ((), ())),
                preferred_element_type=jnp.float32)
            e = jnp.where(e >= 0, e, NEG_SLOPE * e)
            acc = jnp.where(adjb == k + 1, e, acc)
        m = jnp.max(acc, axis=1, keepdims=True)
        p = jnp.exp(acc - m)
        s = jnp.sum(p, axis=1, keepdims=True)
        w = (p / s).astype(jnp.bfloat16)
        out_ref[b] = jax.lax.dot_general(
            w, hb, (((1,), (0,)), ((), ())),
            preferred_element_type=jnp.float32)


def kernel(total_items, total_adj, embedding, a_0, a_1, a_2, a_3):
    idx = total_items.reshape(1, B * L).astype(jnp.int32)
    h = _sc_gather(embedding, idx).reshape(B, L, D)
    a4 = jnp.concatenate([a_0.T, a_1.T, a_2.T, a_3.T], axis=0)
    return pl.pallas_call(
        _attn_body,
        grid=(B // BB,),
        in_specs=[
            pl.BlockSpec((4, D), lambda i: (0, 0)),
            pl.BlockSpec((BB, L, D), lambda i: (i, 0, 0)),
            pl.BlockSpec((BB, L, L), lambda i: (i, 0, 0)),
        ],
        out_specs=pl.BlockSpec((BB, L, D), lambda i: (i, 0, 0)),
        out_shape=jax.ShapeDtypeStruct((B, L, D), jnp.float32),
    )(a4, h, total_adj)
